# baseline (speedup 1.0000x reference)
import jax
import jax.numpy as jnp
from jax import lax
from jax.experimental import pallas as pl
from jax.experimental.pallas import tpu as pltpu
from jax.experimental.pallas import tpu_sc as plsc

B = 64
T = 4096
NCOL = T // 16
NCHUNK = NCOL // 16
N_STEPS = 20
LR = 0.05

NC, NS = 2, 16
ROWS_PER_W = B // (NC * NS)


def _compute_row(mv, sv, m4s_v, s4s_v, m4m_v, s4m_v,
                 m16s_v, s16s_v, m16m_v, s16m_v, nb_v, exo, sdo):
    def nconst(k):
        return nb_v[pl.ds(16 * k, 16)]

    n1_0, n1_1 = nconst(0), nconst(1)
    n4_0, n4_1 = nconst(2), nconst(3)
    n16_0, n16_1 = nconst(4), nconst(5)
    s1 = n1_1 + 0.5
    o1 = n1_0
    r4 = 1.0 / (n4_1 + 0.5)
    r16 = 1.0 / (n16_1 + 0.5)
    A16 = s1 * r16 * (1.0 / 16.0)
    C16 = (o1 - n16_0) * r16
    D16 = s1 * r16 * (1.0 / 340.0)
    E16 = 0.0 - n16_0 * r16
    A4 = s1 * r4 * 0.25
    C4 = (o1 - n4_0) * r4
    D4 = s1 * r4 * 0.2
    E4 = 0.0 - n4_0 * r4
    nLRA16 = (0.0 - LR) * A16
    nLRD16 = (0.0 - LR) * D16
    nLRA4 = (0.0 - LR) * A4
    nLRD4 = (0.0 - LR) * D4

    @plsc.parallel_loop(0, NCHUNK, step=1, unroll=2)
    def chunk_body(c):
        iota = lax.iota(jnp.int32, 16)
        gbase = c * 256 + iota * 16
        g4base = c * 64 + iota * 4
        m1 = [plsc.load_gather(mv, [gbase + j]) for j in range(16)]
        d = []
        for j in range(16):
            sd = jnp.abs(plsc.load_gather(sv, [gbase + j])) + 0.5
            plsc.store_scatter(sdo, [gbase + j], sd)
            d.append(1.0 - LR / (sd * sd))

        Sm4 = [m1[4 * g] + m1[4 * g + 1] + m1[4 * g + 2] + m1[4 * g + 3]
               for g in range(4)]
        Sm16 = (Sm4[0] + Sm4[1]) + (Sm4[2] + Sm4[3])
        Wm4 = [1.5 * (m1[4 * g + 3] - m1[4 * g]) + 0.5 * (m1[4 * g + 2] - m1[4 * g + 1])
               for g in range(4)]
        Wm16 = ((Wm4[0] + Wm4[1]) + (Wm4[2] + Wm4[3])
                + (-6.0 * Sm4[0] - 2.0 * Sm4[1] + 2.0 * Sm4[2] + 6.0 * Sm4[3]))

        def coeff16(ref_m, ref_s):
            sx = jnp.abs(ref_s[pl.ds(c * 16, 16)]) + 0.5
            ivx = 1.0 / (sx * sx)
            return ivx, ref_m[pl.ds(c * 16, 16)] * ivx

        def coeff4(ref_m, ref_s, g):
            sx = jnp.abs(plsc.load_gather(ref_s, [g4base + g])) + 0.5
            ivx = 1.0 / (sx * sx)
            return ivx, plsc.load_gather(ref_m, [g4base + g]) * ivx

        ivm16, pm16 = coeff16(m16m_v, s16m_v)
        P16m = nLRA16 * A16 * ivm16
        Q16m = nLRA16 * ((Sm16 * A16 + C16) * ivm16 - pm16)
        ivs16, ps16 = coeff16(m16s_v, s16s_v)
        P16s = nLRD16 * D16 * ivs16
        Q16s = nLRD16 * ((Wm16 * D16 + E16) * ivs16 - ps16)
        P4m, P4s, QB, QS = [], [], [], []
        for g in range(4):
            ivm4, pm4 = coeff4(m4m_v, s4m_v, g)
            P4m.append(nLRA4 * A4 * ivm4)
            q4m = nLRA4 * ((Sm4[g] * A4 + C4) * ivm4 - pm4)
            ivs4, ps4 = coeff4(m4s_v, s4s_v, g)
            P4s.append(nLRD4 * D4 * ivs4)
            q4s = nLRD4 * ((Wm4[g] * D4 + E4) * ivs4 - ps4)
            QB.append((Q16m + q4m) + (4.0 * g - 6.0) * Q16s)
            QS.append(Q16s + q4s)

        zero = jnp.zeros((16,), jnp.float32)

        def step(zs):
            S4 = [zs[4 * g] + zs[4 * g + 1] + zs[4 * g + 2] + zs[4 * g + 3]
                  for g in range(4)]
            S16 = (S4[0] + S4[1]) + (S4[2] + S4[3])
            W4 = [1.5 * (zs[4 * g + 3] - zs[4 * g]) + 0.5 * (zs[4 * g + 2] - zs[4 * g + 1])
                  for g in range(4)]
            W16 = ((W4[0] + W4[1]) + (W4[2] + W4[3])
                   + (-6.0 * S4[0] - 2.0 * S4[1] + 2.0 * S4[2] + 6.0 * S4[3]))
            u = S16 * P16m
            v = W16 * P16s
            new = []
            for g in range(4):
                bg = ((S4[g] * P4m[g] + QB[g]) + u) + (4.0 * g - 6.0) * v
                sg = (W4[g] * P4s[g] + QS[g]) + v
                for q in range(4):
                    j = 4 * g + q
                    cc = bg + (q - 1.5) * sg
                    new.append(zs[j] * d[j] + cc)
            return tuple(new)

        def step_body(_, zs):
            return step(step(zs))

        zf = lax.fori_loop(0, N_STEPS // 2, step_body, (zero,) * 16)
        for j in range(16):
            plsc.store_scatter(exo, [gbase + j], zf[j] + m1[j])


def _sc_body(means1, stds1, m4s, s4s, m4m, s4m, m16s, s16s, m16m, s16m, normb,
             ex_out, sd_out, bufs0, bufs1, out0, out1,
             sem_i0, sem_i1, sem_o0, sem_o1):
    wid = lax.axis_index("s") * NC + lax.axis_index("c")
    b0 = wid * ROWS_PER_W
    b1 = b0 + 1
    ins = (means1, stds1, m4s, s4s, m4m, s4m, m16s, s16s, m16m, s16m, normb)

    def fetch(b, bufs, sem):
        return [pltpu.async_copy(src.at[b], dst, sem)
                for src, dst in zip(ins, bufs)]

    cps0 = fetch(b0, bufs0, sem_i0)
    cps1 = fetch(b1, bufs1, sem_i1)
    for cp in cps0:
        cp.wait()
    _compute_row(*bufs0, *out0)
    w0 = [pltpu.async_copy(out0[0], ex_out.at[b0], sem_o0),
          pltpu.async_copy(out0[1], sd_out.at[b0], sem_o0)]
    for cp in cps1:
        cp.wait()
    _compute_row(*bufs1, *out1)
    w1 = [pltpu.async_copy(out1[0], ex_out.at[b1], sem_o1),
          pltpu.async_copy(out1[1], sd_out.at[b1], sem_o1)]
    for w in w0 + w1:
        w.wait()


def _row_bufs():
    return [
        pltpu.VMEM((T,), jnp.float32),
        pltpu.VMEM((T,), jnp.float32),
        pltpu.VMEM((T // 4,), jnp.float32),
        pltpu.VMEM((T // 4,), jnp.float32),
        pltpu.VMEM((T // 4,), jnp.float32),
        pltpu.VMEM((T // 4,), jnp.float32),
        pltpu.VMEM((NCOL,), jnp.float32),
        pltpu.VMEM((NCOL,), jnp.float32),
        pltpu.VMEM((NCOL,), jnp.float32),
        pltpu.VMEM((NCOL,), jnp.float32),
        pltpu.VMEM((96,), jnp.float32),
    ]


@jax.jit
def _run(means1, stds1, m4s, s4s, m4m, s4m, m16s, s16s, m16m, s16m, normb):
    f = pl.kernel(
        _sc_body,
        out_type=(jax.ShapeDtypeStruct((B, T), jnp.float32),
                  jax.ShapeDtypeStruct((B, T), jnp.float32)),
        mesh=plsc.VectorSubcoreMesh(
            core_axis_name="c", subcore_axis_name="s",
            num_cores=NC, num_subcores=NS),
        scratch_types=[
            _row_bufs(),
            _row_bufs(),
            [pltpu.VMEM((T,), jnp.float32), pltpu.VMEM((T,), jnp.float32)],
            [pltpu.VMEM((T,), jnp.float32), pltpu.VMEM((T,), jnp.float32)],
            pltpu.SemaphoreType.DMA,
            pltpu.SemaphoreType.DMA,
            pltpu.SemaphoreType.DMA,
            pltpu.SemaphoreType.DMA,
        ],
        compiler_params=pltpu.CompilerParams(needs_layout_passes=False),
    )
    return f(means1, stds1, m4s, s4s, m4m, s4m, m16s, s16s, m16m, s16m, normb)


def kernel(means1, stds1, means4_slope, stds4_slope, means4_sum, stds4_sum,
           means16_slope, stds16_slope, means16_sum, stds16_sum,
           norm1, norm4, norm16):
    normb = jnp.repeat(
        jnp.concatenate([norm1, norm4, norm16], axis=1), 16, axis=1)
    ex_final, all_preds_std = _run(
        means1, stds1, means4_slope, stds4_slope, means4_sum, stds4_sum,
        means16_slope, stds16_slope, means16_sum, stds16_sum, normb)
    return ex_final, all_preds_std


# speedup vs baseline: 1.0281x; 1.0281x over previous
import jax
import jax.numpy as jnp
from jax import lax
from jax.experimental import pallas as pl
from jax.experimental.pallas import tpu as pltpu
from jax.experimental.pallas import tpu_sc as plsc

B = 64
T = 4096
NCOL = T // 16
NCHUNK = NCOL // 16
N_STEPS = 20
LR = 0.05

NC, NS = 2, 16
ROWS_PER_W = B // (NC * NS)


def _compute_row(mv, sv, m4s_v, s4s_v, m4m_v, s4m_v,
                 m16s_v, s16s_v, m16m_v, s16m_v, nb_v, exo, sdo):
    def nconst(k):
        return nb_v[pl.ds(16 * k, 16)]

    n1_0, n1_1 = nconst(0), nconst(1)
    n4_0, n4_1 = nconst(2), nconst(3)
    n16_0, n16_1 = nconst(4), nconst(5)
    s1 = n1_1 + 0.5
    o1 = n1_0
    r4 = 1.0 / (n4_1 + 0.5)
    r16 = 1.0 / (n16_1 + 0.5)
    A16 = s1 * r16 * (1.0 / 16.0)
    C16 = (o1 - n16_0) * r16
    D16 = s1 * r16 * (1.0 / 340.0)
    E16 = 0.0 - n16_0 * r16
    A4 = s1 * r4 * 0.25
    C4 = (o1 - n4_0) * r4
    D4 = s1 * r4 * 0.2
    E4 = 0.0 - n4_0 * r4
    nLRA16 = (0.0 - LR) * A16
    nLRD16 = (0.0 - LR) * D16
    nLRA4 = (0.0 - LR) * A4
    nLRD4 = (0.0 - LR) * D4

    def chunk_body(c, carry):
        iota = lax.iota(jnp.int32, 16)
        gbase = c * 256 + iota * 16
        g4base = c * 64 + iota * 4
        m1 = [plsc.load_gather(mv, [gbase + j]) for j in range(16)]
        d = []
        for j in range(16):
            sd = jnp.abs(plsc.load_gather(sv, [gbase + j])) + 0.5
            plsc.store_scatter(sdo, [gbase + j], sd)
            d.append(1.0 - LR / (sd * sd))

        Sm4 = [m1[4 * g] + m1[4 * g + 1] + m1[4 * g + 2] + m1[4 * g + 3]
               for g in range(4)]
        Sm16 = (Sm4[0] + Sm4[1]) + (Sm4[2] + Sm4[3])
        Wm4 = [1.5 * (m1[4 * g + 3] - m1[4 * g]) + 0.5 * (m1[4 * g + 2] - m1[4 * g + 1])
               for g in range(4)]
        Wm16 = ((Wm4[0] + Wm4[1]) + (Wm4[2] + Wm4[3])
                + (-6.0 * Sm4[0] - 2.0 * Sm4[1] + 2.0 * Sm4[2] + 6.0 * Sm4[3]))

        def coeff16(ref_m, ref_s):
            sx = jnp.abs(ref_s[pl.ds(c * 16, 16)]) + 0.5
            ivx = 1.0 / (sx * sx)
            return ivx, ref_m[pl.ds(c * 16, 16)] * ivx

        def coeff4(ref_m, ref_s, g):
            sx = jnp.abs(plsc.load_gather(ref_s, [g4base + g])) + 0.5
            ivx = 1.0 / (sx * sx)
            return ivx, plsc.load_gather(ref_m, [g4base + g]) * ivx

        ivm16, pm16 = coeff16(m16m_v, s16m_v)
        P16m = nLRA16 * A16 * ivm16
        Q16m = nLRA16 * ((Sm16 * A16 + C16) * ivm16 - pm16)
        ivs16, ps16 = coeff16(m16s_v, s16s_v)
        P16s = nLRD16 * D16 * ivs16
        Q16s = nLRD16 * ((Wm16 * D16 + E16) * ivs16 - ps16)
        P4m, P4s, QB, QS = [], [], [], []
        for g in range(4):
            ivm4, pm4 = coeff4(m4m_v, s4m_v, g)
            P4m.append(nLRA4 * A4 * ivm4)
            q4m = nLRA4 * ((Sm4[g] * A4 + C4) * ivm4 - pm4)
            ivs4, ps4 = coeff4(m4s_v, s4s_v, g)
            P4s.append(nLRD4 * D4 * ivs4)
            q4s = nLRD4 * ((Wm4[g] * D4 + E4) * ivs4 - ps4)
            QB.append((Q16m + q4m) + (4.0 * g - 6.0) * Q16s)
            QS.append(Q16s + q4s)

        zero = jnp.zeros((16,), jnp.float32)

        def step(zs):
            S4 = [zs[4 * g] + zs[4 * g + 1] + zs[4 * g + 2] + zs[4 * g + 3]
                  for g in range(4)]
            S16 = (S4[0] + S4[1]) + (S4[2] + S4[3])
            W4 = [1.5 * (zs[4 * g + 3] - zs[4 * g]) + 0.5 * (zs[4 * g + 2] - zs[4 * g + 1])
                  for g in range(4)]
            W16 = ((W4[0] + W4[1]) + (W4[2] + W4[3])
                   + (-6.0 * S4[0] - 2.0 * S4[1] + 2.0 * S4[2] + 6.0 * S4[3]))
            u = S16 * P16m
            v = W16 * P16s
            new = []
            for g in range(4):
                bg = ((S4[g] * P4m[g] + QB[g]) + u) + (4.0 * g - 6.0) * v
                sg = (W4[g] * P4s[g] + QS[g]) + v
                for q in range(4):
                    j = 4 * g + q
                    cc = bg + (q - 1.5) * sg
                    new.append(zs[j] * d[j] + cc)
            return tuple(new)

        def step_body(_, zs):
            return step(step(zs))

        zf = lax.fori_loop(0, N_STEPS // 2, step_body, (zero,) * 16)
        for j in range(16):
            plsc.store_scatter(exo, [gbase + j], zf[j] + m1[j])
        return carry

    lax.fori_loop(0, NCHUNK, chunk_body, 0)


def _sc_body(means1, stds1, m4s, s4s, m4m, s4m, m16s, s16s, m16m, s16m, normb,
             ex_out, sd_out, bufs0, bufs1, out0, out1,
             sem_i0, sem_i1, sem_o0, sem_o1):
    wid = lax.axis_index("s") * NC + lax.axis_index("c")
    b0 = wid * ROWS_PER_W
    b1 = b0 + 1
    ins = (means1, stds1, m4s, s4s, m4m, s4m, m16s, s16s, m16m, s16m, normb)

    def fetch(b, bufs, sem):
        return [pltpu.async_copy(src.at[b], dst, sem)
                for src, dst in zip(ins, bufs)]

    cps0 = fetch(b0, bufs0, sem_i0)
    cps1 = fetch(b1, bufs1, sem_i1)
    for cp in cps0:
        cp.wait()
    _compute_row(*bufs0, *out0)
    w0 = [pltpu.async_copy(out0[0], ex_out.at[b0], sem_o0),
          pltpu.async_copy(out0[1], sd_out.at[b0], sem_o0)]
    for cp in cps1:
        cp.wait()
    _compute_row(*bufs1, *out1)
    w1 = [pltpu.async_copy(out1[0], ex_out.at[b1], sem_o1),
          pltpu.async_copy(out1[1], sd_out.at[b1], sem_o1)]
    for w in w0 + w1:
        w.wait()


def _row_bufs():
    return [
        pltpu.VMEM((T,), jnp.float32),
        pltpu.VMEM((T,), jnp.float32),
        pltpu.VMEM((T // 4,), jnp.float32),
        pltpu.VMEM((T // 4,), jnp.float32),
        pltpu.VMEM((T // 4,), jnp.float32),
        pltpu.VMEM((T // 4,), jnp.float32),
        pltpu.VMEM((NCOL,), jnp.float32),
        pltpu.VMEM((NCOL,), jnp.float32),
        pltpu.VMEM((NCOL,), jnp.float32),
        pltpu.VMEM((NCOL,), jnp.float32),
        pltpu.VMEM((96,), jnp.float32),
    ]


@jax.jit
def _run(means1, stds1, m4s, s4s, m4m, s4m, m16s, s16s, m16m, s16m, normb):
    f = pl.kernel(
        _sc_body,
        out_type=(jax.ShapeDtypeStruct((B, T), jnp.float32),
                  jax.ShapeDtypeStruct((B, T), jnp.float32)),
        mesh=plsc.VectorSubcoreMesh(
            core_axis_name="c", subcore_axis_name="s",
            num_cores=NC, num_subcores=NS),
        scratch_types=[
            _row_bufs(),
            _row_bufs(),
            [pltpu.VMEM((T,), jnp.float32), pltpu.VMEM((T,), jnp.float32)],
            [pltpu.VMEM((T,), jnp.float32), pltpu.VMEM((T,), jnp.float32)],
            pltpu.SemaphoreType.DMA,
            pltpu.SemaphoreType.DMA,
            pltpu.SemaphoreType.DMA,
            pltpu.SemaphoreType.DMA,
        ],
        compiler_params=pltpu.CompilerParams(needs_layout_passes=False),
    )
    return f(means1, stds1, m4s, s4s, m4m, s4m, m16s, s16s, m16m, s16m, normb)


def kernel(means1, stds1, means4_slope, stds4_slope, means4_sum, stds4_sum,
           means16_slope, stds16_slope, means16_sum, stds16_sum,
           norm1, norm4, norm16):
    normb = jnp.repeat(
        jnp.concatenate([norm1, norm4, norm16], axis=1), 16, axis=1)
    ex_final, all_preds_std = _run(
        means1, stds1, means4_slope, stds4_slope, means4_sum, stds4_sum,
        means16_slope, stds16_slope, means16_sum, stds16_sum, normb)
    return ex_final, all_preds_std
